# baseline (speedup 1.0000x reference)
import functools

import jax
import jax.numpy as jnp
from jax.experimental import pallas as pl
from jax.experimental.pallas import tpu as pltpu

_G = 64
_BLK = 2000


def _fused_body(nblk, x1_ref, bat1_hbm, x2_ref, bat2_hbm, w1_hbm, b1_hbm,
                w2_hbm, b2_hbm, fcw_hbm, fcb_hbm, out_ref,
                s1_ref, c1_ref, s2_ref, c2_ref, st1_ref, st2_ref,
                wcat_ref, bcat_ref, stcat_ref, ccat_ref,
                w1_vm, w2_vm, b1_vm, b2_vm,
                fcw_vm, fcb_vm, bat1_vm, bat2_vm, sems):
    i = pl.program_id(0)
    blk = x1_ref.shape[0]
    n = bat1_vm.shape[0]
    f = w1_vm.shape[0]
    h = w1_vm.shape[1]
    gids = jax.lax.broadcasted_iota(jnp.int32, (_G, 1), 0)

    @pl.when(i == 0)
    def _init():
        copies = [
            pltpu.make_async_copy(bat1_hbm, bat1_vm, sems.at[0]),
            pltpu.make_async_copy(bat2_hbm, bat2_vm, sems.at[1]),
            pltpu.make_async_copy(w1_hbm, w1_vm, sems.at[2]),
            pltpu.make_async_copy(w2_hbm, w2_vm, sems.at[3]),
            pltpu.make_async_copy(b1_hbm, b1_vm, sems.at[4]),
            pltpu.make_async_copy(b2_hbm, b2_vm, sems.at[5]),
            pltpu.make_async_copy(fcw_hbm, fcw_vm, sems.at[6]),
            pltpu.make_async_copy(fcb_hbm, fcb_vm, sems.at[7]),
        ]
        for cp in copies:
            cp.start()
        s1_ref[...] = jnp.zeros_like(s1_ref)
        s2_ref[...] = jnp.zeros_like(s2_ref)
        tri = (jax.lax.broadcasted_iota(jnp.int32, (_G, _G), 1)
               < jax.lax.broadcasted_iota(jnp.int32, (_G, _G), 0)
               ).astype(jnp.float32)
        copies[0].wait()
        cnt1 = jnp.sum((bat1_vm[...].reshape(1, n) == gids).astype(jnp.float32),
                       axis=1, keepdims=True)
        c1_ref[...] = cnt1
        st1_ref[...] = jnp.dot(tri, cnt1, preferred_element_type=jnp.float32)
        copies[1].wait()
        cnt2 = jnp.sum((bat2_vm[...].reshape(1, n) == gids).astype(jnp.float32),
                       axis=1, keepdims=True)
        c2_ref[...] = cnt2
        st2_ref[...] = jnp.dot(tri, cnt2, preferred_element_type=jnp.float32)
        for cp in copies[2:]:
            cp.wait()
        wcat_ref[...] = jnp.zeros_like(wcat_ref)
        wcat_ref[:f, :h] = w1_vm[...].astype(jnp.bfloat16)
        wcat_ref[f:, h:] = w2_vm[...].astype(jnp.bfloat16)
        bcat_ref[:1, :h] = b1_vm[...].reshape(1, -1).astype(jnp.bfloat16)
        bcat_ref[:1, h:] = b2_vm[...].reshape(1, -1).astype(jnp.bfloat16)
        stcat_ref[:_G, :] = st1_ref[...]
        stcat_ref[_G:, :] = st2_ref[...]
        ccat_ref[:_G, :] = c1_ref[...]
        ccat_ref[_G:, :] = c2_ref[...]

    half = blk // 2
    ds1 = jnp.zeros((_G, h), jnp.float32)
    ds2 = jnp.zeros((_G, h), jnp.float32)
    for p in range(2):
        sl = pl.ds(p * half, half)
        rows = (i * blk + p * half
                + jax.lax.broadcasted_iota(jnp.int32, (1, half), 1)
                ).astype(jnp.float32)
        xcat = jnp.concatenate([x1_ref[sl, :].astype(jnp.bfloat16),
                                x2_ref[sl, :].astype(jnp.bfloat16)],
                               axis=1)
        hcat = jnp.maximum(
            jnp.dot(xcat, wcat_ref[...], preferred_element_type=jnp.float32)
            + bcat_ref[...].astype(jnp.float32),
            0.0).astype(jnp.bfloat16)
        start = stcat_ref[...]
        stop = start + ccat_ref[...]
        onehot_t = ((rows >= start) & (rows < stop)).astype(jnp.bfloat16)
        ds = jnp.dot(onehot_t, hcat, preferred_element_type=jnp.float32)
        ds1 = ds1 + ds[:_G, :h]
        ds2 = ds2 + ds[_G:, h:]
    s1_ref[...] += ds1
    s2_ref[...] += ds2

    @pl.when(i == nblk - 1)
    def _finish():
        p1 = s1_ref[...] / jnp.maximum(c1_ref[...], 1.0)
        p2 = s2_ref[...] / jnp.maximum(c2_ref[...], 1.0)
        pool = (p1 + p2) * 0.5
        out_ref[...] = (jnp.dot(pool, fcw_vm[...],
                                preferred_element_type=jnp.float32)
                        + fcb_vm[...].reshape(1, -1))


@jax.jit
def _run(x1, bat1, x2, bat2, W1, b1, W2, b2, fcW, fcb):
    n, f1 = x1.shape
    h = W1.shape[1]
    out_dim = fcW.shape[1]
    nblk = n // _BLK

    row_spec = pl.BlockSpec((_BLK, f1), lambda i: (i, 0))
    hbm_spec = pl.BlockSpec(memory_space=pl.ANY)

    return pl.pallas_call(
        functools.partial(_fused_body, nblk),
        grid=(nblk,),
        in_specs=[row_spec, hbm_spec, row_spec, hbm_spec,
                  hbm_spec, hbm_spec, hbm_spec, hbm_spec,
                  hbm_spec, hbm_spec],
        out_specs=pl.BlockSpec((_G, out_dim), lambda i: (0, 0)),
        out_shape=jax.ShapeDtypeStruct((_G, out_dim), jnp.float32),
        scratch_shapes=[
            pltpu.VMEM((_G, h), jnp.float32),
            pltpu.VMEM((_G, 1), jnp.float32),
            pltpu.VMEM((_G, h), jnp.float32),
            pltpu.VMEM((_G, 1), jnp.float32),
            pltpu.VMEM((_G, 1), jnp.float32),
            pltpu.VMEM((_G, 1), jnp.float32),
            pltpu.VMEM((2 * f1, 2 * h), jnp.bfloat16),
            pltpu.VMEM((1, 2 * h), jnp.bfloat16),
            pltpu.VMEM((2 * _G, 1), jnp.float32),
            pltpu.VMEM((2 * _G, 1), jnp.float32),
            pltpu.VMEM((f1, h), jnp.float32),
            pltpu.VMEM((f1, h), jnp.float32),
            pltpu.VMEM((h,), jnp.float32),
            pltpu.VMEM((h,), jnp.float32),
            pltpu.VMEM((h, out_dim), jnp.float32),
            pltpu.VMEM((out_dim,), jnp.float32),
            pltpu.VMEM((n,), jnp.int32),
            pltpu.VMEM((n,), jnp.int32),
            pltpu.SemaphoreType.DMA((8,)),
        ],
    )(x1, bat1, x2, bat2, W1, b1, W2, b2, fcW, fcb)


def kernel(x1, edge_index1, edge_attr1, batch1, x2, edge_index2, edge_attr2,
           batch2, W1, b1, W2, b2, fcW, fcb):
    del edge_index1, edge_attr1, edge_index2, edge_attr2
    return _run(x1, batch1, x2, batch2, W1, b1, W2, b2, fcW, fcb)


# speedup vs baseline: 1.0764x; 1.0764x over previous
import functools

import jax
import jax.numpy as jnp
from jax.experimental import pallas as pl
from jax.experimental.pallas import tpu as pltpu

_G = 64
_BLK = 2000


def _fused_body(nblk, x1_ref, bat1_hbm, x2_ref, bat2_hbm, w1_hbm, b1_hbm,
                w2_hbm, b2_hbm, fcw_hbm, fcb_hbm, out_ref,
                s1_ref, c1_ref, s2_ref, c2_ref, st1_ref, st2_ref,
                wcat_ref, bcat_ref, stcat_ref, ccat_ref,
                w1_vm, w2_vm, b1_vm, b2_vm,
                fcw_vm, fcb_vm, bat1_vm, bat2_vm, sems):
    i = pl.program_id(0)
    blk = x1_ref.shape[0]
    n = bat1_vm.shape[0]
    f = w1_vm.shape[0]
    h = w1_vm.shape[1]
    gids = jax.lax.broadcasted_iota(jnp.int32, (_G, 1), 0)

    @pl.when(i == 0)
    def _init():
        copies = [
            pltpu.make_async_copy(bat1_hbm, bat1_vm, sems.at[0]),
            pltpu.make_async_copy(bat2_hbm, bat2_vm, sems.at[1]),
            pltpu.make_async_copy(w1_hbm, w1_vm, sems.at[2]),
            pltpu.make_async_copy(w2_hbm, w2_vm, sems.at[3]),
            pltpu.make_async_copy(b1_hbm, b1_vm, sems.at[4]),
            pltpu.make_async_copy(b2_hbm, b2_vm, sems.at[5]),
            pltpu.make_async_copy(fcw_hbm, fcw_vm, sems.at[6]),
            pltpu.make_async_copy(fcb_hbm, fcb_vm, sems.at[7]),
        ]
        for cp in copies:
            cp.start()
        s1_ref[...] = jnp.zeros_like(s1_ref)
        s2_ref[...] = jnp.zeros_like(s2_ref)
        tri = (jax.lax.broadcasted_iota(jnp.int32, (_G, _G), 1)
               < jax.lax.broadcasted_iota(jnp.int32, (_G, _G), 0)
               ).astype(jnp.float32)
        copies[0].wait()
        cnt1 = jnp.sum((bat1_vm[...].reshape(1, n) == gids).astype(jnp.float32),
                       axis=1, keepdims=True)
        c1_ref[...] = cnt1
        st1_ref[...] = jnp.dot(tri, cnt1, preferred_element_type=jnp.float32)
        copies[1].wait()
        cnt2 = jnp.sum((bat2_vm[...].reshape(1, n) == gids).astype(jnp.float32),
                       axis=1, keepdims=True)
        c2_ref[...] = cnt2
        st2_ref[...] = jnp.dot(tri, cnt2, preferred_element_type=jnp.float32)
        for cp in copies[2:]:
            cp.wait()
        wcat_ref[...] = jnp.zeros_like(wcat_ref)
        wcat_ref[:f, :h] = w1_vm[...].astype(jnp.bfloat16)
        wcat_ref[f:, h:] = w2_vm[...].astype(jnp.bfloat16)
        bcat_ref[:1, :h] = b1_vm[...].reshape(1, -1).astype(jnp.bfloat16)
        bcat_ref[:1, h:] = b2_vm[...].reshape(1, -1).astype(jnp.bfloat16)
        stcat_ref[:_G, :] = st1_ref[...]
        stcat_ref[_G:, :] = st2_ref[...]
        ccat_ref[:_G, :] = c1_ref[...]
        ccat_ref[_G:, :] = c2_ref[...]

    half = blk // 2
    ds1 = jnp.zeros((_G, h), jnp.float32)
    ds2 = jnp.zeros((_G, h), jnp.float32)
    for p in range(2):
        sl = pl.ds(p * half, half)
        rows = (i * blk + p * half
                + jax.lax.broadcasted_iota(jnp.int32, (1, half), 1)
                ).astype(jnp.float32)
        xcat = jnp.concatenate([x1_ref[sl, :].astype(jnp.bfloat16),
                                x2_ref[sl, :].astype(jnp.bfloat16)],
                               axis=1)
        hcat = jnp.maximum(
            jnp.dot(xcat, wcat_ref[...], preferred_element_type=jnp.float32)
            + bcat_ref[...].astype(jnp.float32),
            0.0).astype(jnp.bfloat16)

        def pool(hpart, c_ref, st_ref):
            start = st_ref[...]
            stop = start + c_ref[...]
            onehot_t = ((rows >= start) & (rows < stop)).astype(jnp.bfloat16)
            return jnp.dot(onehot_t, hpart, preferred_element_type=jnp.float32)

        ds1 = ds1 + pool(hcat[:, :h], c1_ref, st1_ref)
        ds2 = ds2 + pool(hcat[:, h:], c2_ref, st2_ref)
    s1_ref[...] += ds1
    s2_ref[...] += ds2

    @pl.when(i == nblk - 1)
    def _finish():
        p1 = s1_ref[...] / jnp.maximum(c1_ref[...], 1.0)
        p2 = s2_ref[...] / jnp.maximum(c2_ref[...], 1.0)
        pool = (p1 + p2) * 0.5
        out_ref[...] = (jnp.dot(pool, fcw_vm[...],
                                preferred_element_type=jnp.float32)
                        + fcb_vm[...].reshape(1, -1))


@jax.jit
def _run(x1, bat1, x2, bat2, W1, b1, W2, b2, fcW, fcb):
    n, f1 = x1.shape
    h = W1.shape[1]
    out_dim = fcW.shape[1]
    nblk = n // _BLK

    row_spec = pl.BlockSpec((_BLK, f1), lambda i: (i, 0))
    hbm_spec = pl.BlockSpec(memory_space=pl.ANY)

    return pl.pallas_call(
        functools.partial(_fused_body, nblk),
        grid=(nblk,),
        in_specs=[row_spec, hbm_spec, row_spec, hbm_spec,
                  hbm_spec, hbm_spec, hbm_spec, hbm_spec,
                  hbm_spec, hbm_spec],
        out_specs=pl.BlockSpec((_G, out_dim), lambda i: (0, 0)),
        out_shape=jax.ShapeDtypeStruct((_G, out_dim), jnp.float32),
        scratch_shapes=[
            pltpu.VMEM((_G, h), jnp.float32),
            pltpu.VMEM((_G, 1), jnp.float32),
            pltpu.VMEM((_G, h), jnp.float32),
            pltpu.VMEM((_G, 1), jnp.float32),
            pltpu.VMEM((_G, 1), jnp.float32),
            pltpu.VMEM((_G, 1), jnp.float32),
            pltpu.VMEM((2 * f1, 2 * h), jnp.bfloat16),
            pltpu.VMEM((1, 2 * h), jnp.bfloat16),
            pltpu.VMEM((2 * _G, 1), jnp.float32),
            pltpu.VMEM((2 * _G, 1), jnp.float32),
            pltpu.VMEM((f1, h), jnp.float32),
            pltpu.VMEM((f1, h), jnp.float32),
            pltpu.VMEM((h,), jnp.float32),
            pltpu.VMEM((h,), jnp.float32),
            pltpu.VMEM((h, out_dim), jnp.float32),
            pltpu.VMEM((out_dim,), jnp.float32),
            pltpu.VMEM((n,), jnp.int32),
            pltpu.VMEM((n,), jnp.int32),
            pltpu.SemaphoreType.DMA((8,)),
        ],
    )(x1, bat1, x2, bat2, W1, b1, W2, b2, fcW, fcb)


def kernel(x1, edge_index1, edge_attr1, batch1, x2, edge_index2, edge_attr2,
           batch2, W1, b1, W2, b2, fcW, fcb):
    del edge_index1, edge_attr1, edge_index2, edge_attr2
    return _run(x1, batch1, x2, batch2, W1, b1, W2, b2, fcW, fcb)
